# Initial kernel scaffold; baseline (speedup 1.0000x reference)
import jax
import jax.numpy as jnp
from jax.experimental import pallas as pl


def kernel(boxes, scores):
    raise NotImplementedError("write your pallas kernel here")


# speedup vs baseline: 13.2805x; 13.2805x over previous
import jax
import jax.numpy as jnp
from jax import lax
from jax.experimental import pallas as pl
from jax.experimental.pallas import tpu as pltpu
from jax.experimental.pallas import tpu_sc as plsc

_IOU_THRESH = 0.5
_MAX_KEEP = 100

_N = 5000
_NSUB = 16
_PER = 320
_NPAD = _NSUB * _PER
_SLICES = _PER // 16
_TAB = _NSUB * 16

_NEG_INF = float("-inf")


def _nms_body(x1h, y1h, x2h, y2h, sh, outh,
              x1v, y1v, x2v, y2v, msv, outv, stv, rbv, shared):
    wid = lax.axis_index("s")
    base = wid * _PER
    base_f = base.astype(jnp.float32)

    lane = lax.iota(jnp.int32, 16)
    lane_f = lane.astype(jnp.float32)
    zeros16 = jnp.zeros((16,), jnp.float32)

    pltpu.sync_copy(x1h.at[pl.ds(base, _PER)], x1v)
    pltpu.sync_copy(y1h.at[pl.ds(base, _PER)], y1v)
    pltpu.sync_copy(x2h.at[pl.ds(base, _PER)], x2v)
    pltpu.sync_copy(y2h.at[pl.ds(base, _PER)], y2v)
    pltpu.sync_copy(sh.at[pl.ds(base, _PER)], msv)
    for j in range(_SLICES):
        outv[pl.ds(16 * j, 16)] = zeros16

    bv0 = jnp.full((16,), _NEG_INF, jnp.float32)
    bif0 = base_f + lane_f
    for j in range(_SLICES):
        v = msv[pl.ds(16 * j, 16)]
        gi = (base_f + 16.0 * j) + lane_f
        upd = v > bv0
        bv0 = jnp.where(upd, v, bv0)
        bif0 = jnp.where(upd, gi, bif0)

    def round_body(t, carry):
        bv, bif = carry
        lm = jnp.max(bv)
        lif = jnp.min(jnp.where(bv == lm, bif, 1e9))
        li = (lif - base_f).astype(jnp.int32)

        li_vec = jnp.full((16,), li, jnp.int32)
        x1l = plsc.load_gather(x1v, [li_vec])
        y1l = plsc.load_gather(y1v, [li_vec])
        x2l = plsc.load_gather(x2v, [li_vec])
        y2l = plsc.load_gather(y2v, [li_vec])

        st = jnp.where(lane == 0, jnp.full((16,), lm), zeros16)
        st = jnp.where(lane == 1, jnp.full((16,), lif), st)
        st = jnp.where(lane == 2, x1l, st)
        st = jnp.where(lane == 3, y1l, st)
        st = jnp.where(lane == 4, x2l, st)
        st = jnp.where(lane == 5, y2l, st)
        stv[...] = st
        par = lax.rem(t, 2) * _TAB
        pltpu.sync_copy(stv, shared.at[pl.ds(par + wid * 16, 16)])
        plsc.subcore_barrier()

        pltpu.sync_copy(shared.at[pl.ds(par, _TAB)], rbv)
        flat = lane * 16
        vals = plsc.load_gather(rbv, [flat])
        gidxf = plsc.load_gather(rbv, [flat + 1])
        m = jnp.max(vals)
        g_f = jnp.min(jnp.where(vals == m, gidxf, 1e9))
        g_i = g_f.astype(jnp.int32)
        wbase = jnp.full((16,), (g_i // _PER) * 16, jnp.int32)
        x1w = plsc.load_gather(rbv, [wbase + 2])
        y1w = plsc.load_gather(rbv, [wbase + 3])
        x2w = plsc.load_gather(rbv, [wbase + 4])
        y2w = plsc.load_gather(rbv, [wbase + 5])
        valid = m > -1e30

        lidx = g_i - base
        am_owner = (lidx >= 0) & (lidx < _PER)
        lidx_c = jnp.clip(lidx, 0, _PER - 1)
        val_out = jnp.where(valid, m, 0.0)
        plsc.store_scatter(
            outv,
            [jnp.full((16,), lidx_c, jnp.int32)],
            jnp.full((16,), val_out),
            mask=(lane == 0) & am_owner,
        )

        aw = (x2w - x1w) * (y2w - y1w)
        nbv = jnp.full((16,), _NEG_INF, jnp.float32)
        nbif = base_f + lane_f
        for j in range(_SLICES):
            sl = pl.ds(16 * j, 16)
            x1s = x1v[sl]
            y1s = y1v[sl]
            x2s = x2v[sl]
            y2s = y2v[sl]
            ix1 = jnp.maximum(x1s, x1w)
            iy1 = jnp.maximum(y1s, y1w)
            ix2 = jnp.minimum(x2s, x2w)
            iy2 = jnp.minimum(y2s, y2w)
            iw = jnp.maximum(ix2 - ix1, 0.0)
            ih = jnp.maximum(iy2 - iy1, 0.0)
            inter = iw * ih
            areas = (x2s - x1s) * (y2s - y1s)
            union = (aw + areas) - inter
            iou = inter / (union + 1e-6)
            supp = (iou > _IOU_THRESH) & valid
            new = jnp.where(supp, _NEG_INF, msv[sl])
            msv[sl] = new
            gi = (base_f + 16.0 * j) + lane_f
            upd = new > nbv
            nbv = jnp.where(upd, new, nbv)
            nbif = jnp.where(upd, gi, nbif)
        return (nbv, nbif)

    lax.fori_loop(0, _MAX_KEEP, round_body, (bv0, bif0))

    pltpu.sync_copy(outv, outh.at[pl.ds(base, _PER)])


@jax.jit
def kernel(boxes, scores):
    x1 = jnp.pad(boxes[:, 0], (0, _NPAD - _N))
    y1 = jnp.pad(boxes[:, 1], (0, _NPAD - _N))
    x2 = jnp.pad(boxes[:, 2], (0, _NPAD - _N))
    y2 = jnp.pad(boxes[:, 3], (0, _NPAD - _N))
    sp = jnp.pad(scores, (0, _NPAD - _N), constant_values=_NEG_INF)

    nms = pl.kernel(
        _nms_body,
        out_type=jax.ShapeDtypeStruct((_NPAD,), jnp.float32),
        mesh=plsc.VectorSubcoreMesh(
            core_axis_name="c", subcore_axis_name="s", num_cores=1
        ),
        scratch_types=[
            pltpu.VMEM((_PER,), jnp.float32),
            pltpu.VMEM((_PER,), jnp.float32),
            pltpu.VMEM((_PER,), jnp.float32),
            pltpu.VMEM((_PER,), jnp.float32),
            pltpu.VMEM((_PER,), jnp.float32),
            pltpu.VMEM((_PER,), jnp.float32),
            pltpu.VMEM((16,), jnp.float32),
            pltpu.VMEM((_TAB,), jnp.float32),
            pltpu.VMEM_SHARED((2 * _TAB,), jnp.float32),
        ],
        compiler_params=pltpu.CompilerParams(needs_layout_passes=False),
    )
    out = nms(x1, y1, x2, y2, sp)
    return out[:_N]
